# Initial kernel scaffold; baseline (speedup 1.0000x reference)
import jax
import jax.numpy as jnp
from jax.experimental import pallas as pl


def kernel(means, sigmas, values, x):
    raise NotImplementedError("write your pallas kernel here")


# speedup vs baseline: 56.8754x; 56.8754x over previous
import functools

import jax
import jax.numpy as jnp
from jax import lax
from jax.experimental import pallas as pl
from jax.experimental.pallas import tpu as pltpu
from jax.experimental.pallas import tpu_sc as plsc

B = 8
K = 32768
OUT_SIZE = 2048
IN_SIZE = 2048
EPSILON = 1e-6

NC = 2
NS = 16
NW = NC * NS
CH = (B * K) // NW
LANES = 16
ITERS = CH // LANES
WPB = NW // B


@functools.partial(
    pl.kernel,
    out_type=jax.ShapeDtypeStruct((NW * OUT_SIZE,), jnp.float32),
    mesh=plsc.VectorSubcoreMesh(core_axis_name="c", subcore_axis_name="s"),
    compiler_params=pltpu.CompilerParams(needs_layout_passes=False),
    scratch_types=[
        pltpu.VMEM((CH,), jnp.float32),
        pltpu.VMEM((CH,), jnp.float32),
        pltpu.VMEM((CH,), jnp.float32),
        pltpu.VMEM((CH,), jnp.float32),
        pltpu.VMEM((CH,), jnp.float32),
        pltpu.VMEM((IN_SIZE,), jnp.float32),
        pltpu.VMEM((OUT_SIZE,), jnp.float32),
    ],
)
def _sc_hyper(mo_hbm, mi_hbm, so_hbm, si_hbm, v_hbm, x_hbm, out_hbm,
              mo_v, mi_v, so_v, si_v, v_v, x_v, y_v):
    c = lax.axis_index("c")
    s = lax.axis_index("s")
    wid = c * NS + s
    b = wid // WPB
    base = wid * CH

    pltpu.sync_copy(mo_hbm.at[pl.ds(base, CH)], mo_v)
    pltpu.sync_copy(mi_hbm.at[pl.ds(base, CH)], mi_v)
    pltpu.sync_copy(so_hbm.at[pl.ds(base, CH)], so_v)
    pltpu.sync_copy(si_hbm.at[pl.ds(base, CH)], si_v)
    pltpu.sync_copy(v_hbm.at[pl.ds(base, CH)], v_v)
    pltpu.sync_copy(x_hbm.at[pl.ds(b * IN_SIZE, IN_SIZE)], x_v)

    zeros = jnp.zeros((LANES,), jnp.float32)

    def zero_body(j, carry):
        y_v[pl.ds(pl.multiple_of(j * LANES, LANES), LANES)] = zeros
        return carry

    lax.fori_loop(0, OUT_SIZE // LANES, zero_body, 0)

    one_i = jnp.ones((LANES,), jnp.int32)
    zero_i = jnp.zeros((LANES,), jnp.int32)

    def body(i, carry):
        off = pl.multiple_of(i * LANES, LANES)
        mo = mo_v[pl.ds(off, LANES)]
        mi = mi_v[pl.ds(off, LANES)]
        so = so_v[pl.ds(off, LANES)]
        si = si_v[pl.ds(off, LANES)]
        val = v_v[pl.ds(off, LANES)]

        flo_i = mo.astype(jnp.int32)
        flo = flo_i.astype(jnp.float32)
        fli_i = mi.astype(jnp.int32)
        fli = fli_i.astype(jnp.float32)
        ceo_i = flo_i + jnp.where(mo > flo, one_i, zero_i)
        cei_i = fli_i + jnp.where(mi > fli, one_i, zero_i)
        ceo = ceo_i.astype(jnp.float32)
        cei = cei_i.astype(jnp.float32)

        uo = (flo - mo) / so
        wo = (ceo - mo) / so
        ui = (fli - mi) / si
        wi = (cei - mi) / si
        ao = jnp.exp(-0.5 * uo * uo)
        bo = jnp.exp(-0.5 * wo * wo)
        ai = jnp.exp(-0.5 * ui * ui)
        bi = jnp.exp(-0.5 * wi * wi)

        total = (ao + bo) * (ai + bi) + 4.0 * EPSILON
        scale = val / total

        xf = plsc.load_gather(x_v, [fli_i])
        xc = plsc.load_gather(x_v, [cei_i])
        t = ai * xf + bi * xc
        plsc.addupdate_scatter(y_v, [flo_i], scale * ao * t)
        plsc.addupdate_scatter(y_v, [ceo_i], scale * bo * t)
        return carry

    lax.fori_loop(0, ITERS, body, 0)

    pltpu.sync_copy(y_v, out_hbm.at[pl.ds(wid * OUT_SIZE, OUT_SIZE)])


def kernel(means, sigmas, values, x):
    mo = means[:, :, 0].reshape(-1)
    mi = means[:, :, 1].reshape(-1)
    so = sigmas[:, :, 0].reshape(-1)
    si = sigmas[:, :, 1].reshape(-1)
    v = values.reshape(-1)
    xr = x.reshape(-1)
    part = _sc_hyper(mo, mi, so, si, v, xr)
    return part.reshape(B, WPB, OUT_SIZE).sum(axis=1)


# speedup vs baseline: 56.9526x; 1.0014x over previous
import functools

import jax
import jax.numpy as jnp
from jax import lax
from jax.experimental import pallas as pl
from jax.experimental.pallas import tpu as pltpu
from jax.experimental.pallas import tpu_sc as plsc

B = 8
K = 32768
OUT_SIZE = 2048
IN_SIZE = 2048
EPSILON = 1e-6

NC = 2
NS = 16
NW = NC * NS
CH = (B * K) // NW
LANES = 16
ITERS = CH // LANES
WPB = NW // B


@functools.partial(
    pl.kernel,
    out_type=jax.ShapeDtypeStruct((NW * OUT_SIZE,), jnp.float32),
    mesh=plsc.VectorSubcoreMesh(core_axis_name="c", subcore_axis_name="s"),
    compiler_params=pltpu.CompilerParams(needs_layout_passes=False),
    scratch_types=[
        pltpu.VMEM((CH,), jnp.float32),
        pltpu.VMEM((CH,), jnp.float32),
        pltpu.VMEM((CH,), jnp.float32),
        pltpu.VMEM((CH,), jnp.float32),
        pltpu.VMEM((CH,), jnp.float32),
        pltpu.VMEM((IN_SIZE,), jnp.float32),
        pltpu.VMEM((OUT_SIZE,), jnp.float32),
    ],
)
def _sc_hyper(mo_hbm, mi_hbm, so_hbm, si_hbm, v_hbm, x_hbm, out_hbm,
              mo_v, mi_v, so_v, si_v, v_v, x_v, y_v):
    c = lax.axis_index("c")
    s = lax.axis_index("s")
    wid = c * NS + s
    b = wid // WPB
    base = wid * CH

    pltpu.sync_copy(mo_hbm.at[pl.ds(base, CH)], mo_v)
    pltpu.sync_copy(mi_hbm.at[pl.ds(base, CH)], mi_v)
    pltpu.sync_copy(so_hbm.at[pl.ds(base, CH)], so_v)
    pltpu.sync_copy(si_hbm.at[pl.ds(base, CH)], si_v)
    pltpu.sync_copy(v_hbm.at[pl.ds(base, CH)], v_v)
    pltpu.sync_copy(x_hbm.at[pl.ds(b * IN_SIZE, IN_SIZE)], x_v)

    zeros = jnp.zeros((LANES,), jnp.float32)

    def zero_body(j, carry):
        y_v[pl.ds(pl.multiple_of(j * LANES, LANES), LANES)] = zeros
        return carry

    lax.fori_loop(0, OUT_SIZE // LANES, zero_body, 0)

    one_i = jnp.ones((LANES,), jnp.int32)
    zero_i = jnp.zeros((LANES,), jnp.int32)

    def body(i, carry):
        off = pl.multiple_of(i * LANES, LANES)
        mo = mo_v[pl.ds(off, LANES)]
        mi = mi_v[pl.ds(off, LANES)]
        so = so_v[pl.ds(off, LANES)]
        si = si_v[pl.ds(off, LANES)]
        val = v_v[pl.ds(off, LANES)]

        flo_i = mo.astype(jnp.int32)
        flo = flo_i.astype(jnp.float32)
        fli_i = mi.astype(jnp.int32)
        fli = fli_i.astype(jnp.float32)
        ceo_i = flo_i + jnp.where(mo > flo, one_i, zero_i)
        cei_i = fli_i + jnp.where(mi > fli, one_i, zero_i)
        ceo = ceo_i.astype(jnp.float32)
        cei = cei_i.astype(jnp.float32)

        uo = (flo - mo) / so
        wo = (ceo - mo) / so
        ui = (fli - mi) / si
        wi = (cei - mi) / si
        ao = jnp.exp(-0.5 * uo * uo)
        bo = jnp.exp(-0.5 * wo * wo)
        ai = jnp.exp(-0.5 * ui * ui)
        bi = jnp.exp(-0.5 * wi * wi)

        total = (ao + bo) * (ai + bi) + 4.0 * EPSILON
        scale = val / total

        xf = plsc.load_gather(x_v, [fli_i])
        xc = plsc.load_gather(x_v, [cei_i])
        t = ai * xf + bi * xc
        plsc.addupdate_scatter(y_v, [flo_i], scale * ao * t)
        plsc.addupdate_scatter(y_v, [ceo_i], scale * bo * t)
        return carry

    lax.fori_loop(0, ITERS, body, 0, unroll=4)

    pltpu.sync_copy(y_v, out_hbm.at[pl.ds(wid * OUT_SIZE, OUT_SIZE)])


def kernel(means, sigmas, values, x):
    mo = means[:, :, 0].reshape(-1)
    mi = means[:, :, 1].reshape(-1)
    so = sigmas[:, :, 0].reshape(-1)
    si = sigmas[:, :, 1].reshape(-1)
    v = values.reshape(-1)
    xr = x.reshape(-1)
    part = _sc_hyper(mo, mi, so, si, v, xr)
    return part.reshape(B, WPB, OUT_SIZE).sum(axis=1)
